# Initial kernel scaffold; baseline (speedup 1.0000x reference)
import jax
import jax.numpy as jnp
from jax.experimental import pallas as pl


def kernel(x, tables, W1, gamma, beta, W2, b2):
    raise NotImplementedError("write your pallas kernel here")


# speedup vs baseline: 7.5904x; 7.5904x over previous
import functools

import jax
import jax.numpy as jnp
from jax import lax
from jax.experimental import pallas as pl
from jax.experimental.pallas import tpu as pltpu
from jax.experimental.pallas import tpu_sc as plsc

B = 16384
F = 26
DENSE = 13
V = 100000
H = 32
EMB_W = F * H
LOOKUPS = B * F

NC = 2
NS = 16
NW = NC * NS
PER_W = LOOKUPS // NW
CHUNK = 128
NCH = PER_W // CHUNK

BM = 1024
NB = B // BM


def _sc_gather(tab2, idx3):
    mesh = plsc.VectorSubcoreMesh(core_axis_name="c", subcore_axis_name="s")

    @functools.partial(
        pl.kernel,
        mesh=mesh,
        compiler_params=pltpu.CompilerParams(use_tc_tiling_on_sc=False),
        out_type=jax.ShapeDtypeStruct((LOOKUPS, H), jnp.float32),
        scratch_types=[
            pltpu.VMEM((NCH, CHUNK), jnp.int32),
            pltpu.VMEM((CHUNK, H), jnp.float32),
            pltpu.SemaphoreType.DMA,
        ],
    )
    def k(tab_hbm, idx_hbm, out_hbm, idx_v, rows_v, sem):
        wid = lax.axis_index("s") * NC + lax.axis_index("c")
        base = wid * PER_W
        pltpu.sync_copy(idx_hbm.at[wid], idx_v)

        def body(j, carry):
            pltpu.async_copy(tab_hbm.at[idx_v.at[j]], rows_v, sem).wait()
            pltpu.sync_copy(rows_v, out_hbm.at[pl.ds(base + j * CHUNK, CHUNK)])
            return carry

        lax.fori_loop(0, NCH, body, 0)

    return k(tab2, idx3)


def _mlp1_body(emb_ref, xd_ref, w1e_ref, w1d_ref, z_ref, stats_ref):
    j = pl.program_id(0)
    z = jnp.dot(emb_ref[...], w1e_ref[...], preferred_element_type=jnp.float32)
    z = z + jnp.dot(xd_ref[...], w1d_ref[...], preferred_element_type=jnp.float32)
    z_ref[...] = z
    s1 = jnp.sum(z, axis=0, keepdims=True)
    s2 = jnp.sum(z * z, axis=0, keepdims=True)

    @pl.when(j == 0)
    def _():
        stats_ref[...] = jnp.zeros_like(stats_ref)

    stats_ref[...] += jnp.concatenate(
        [s1, s2, jnp.zeros((6, 128), jnp.float32)], axis=0)


def _mlp1(emb, xdp, w1e, w1dp):
    return pl.pallas_call(
        _mlp1_body,
        grid=(NB,),
        in_specs=[
            pl.BlockSpec((BM, EMB_W), lambda j: (j, 0)),
            pl.BlockSpec((BM, 16), lambda j: (j, 0)),
            pl.BlockSpec((EMB_W, 128), lambda j: (0, 0)),
            pl.BlockSpec((16, 128), lambda j: (0, 0)),
        ],
        out_specs=[
            pl.BlockSpec((BM, 128), lambda j: (j, 0)),
            pl.BlockSpec((8, 128), lambda j: (0, 0)),
        ],
        out_shape=[
            jax.ShapeDtypeStruct((B, 128), jnp.float32),
            jax.ShapeDtypeStruct((8, 128), jnp.float32),
        ],
    )(emb, xdp, w1e, w1dp)


def _mlp2_body(z_ref, stats_ref, gb_ref, w2_ref, b2_ref, out_ref):
    stats = stats_ref[...]
    mean = stats[0:1] / B
    var = stats[1:2] / B - mean * mean
    scale = gb_ref[0:1] * lax.rsqrt(var + 1e-5)
    shift = gb_ref[1:2] - mean * scale
    a = jnp.maximum(z_ref[...] * scale + shift, 0.0)
    o = jnp.sum(a * w2_ref[...], axis=1, keepdims=True) + b2_ref[...]
    out_ref[...] = jax.nn.sigmoid(o)


def _mlp2(z, stats, gb, w2row, b2):
    return pl.pallas_call(
        _mlp2_body,
        grid=(NB,),
        in_specs=[
            pl.BlockSpec((BM, 128), lambda j: (j, 0)),
            pl.BlockSpec((8, 128), lambda j: (0, 0)),
            pl.BlockSpec((2, 128), lambda j: (0, 0)),
            pl.BlockSpec((1, 128), lambda j: (0, 0)),
            pl.BlockSpec((1, 1), lambda j: (0, 0)),
        ],
        out_specs=pl.BlockSpec((BM, 1), lambda j: (j, 0)),
        out_shape=jax.ShapeDtypeStruct((B, 1), jnp.float32),
    )(z, stats, gb, w2row, b2)


def kernel(x, tables, W1, gamma, beta, W2, b2):
    idx = x[:, :F].astype(jnp.int32)
    flat_idx = (idx + (jnp.arange(F, dtype=jnp.int32) * V)[None, :]).reshape(-1)
    idx3 = flat_idx.reshape(NW, NCH, CHUNK)
    tab2 = tables.reshape(F * V, H)

    emb = _sc_gather(tab2, idx3).reshape(B, EMB_W)

    xdp = jnp.pad(x[:, F:], ((0, 0), (0, 16 - DENSE)))
    w1e = W1[:EMB_W]
    w1dp = jnp.pad(W1[EMB_W:], ((0, 16 - DENSE), (0, 0)))
    z, stats = _mlp1(emb, xdp, w1e, w1dp)

    gb = jnp.stack([gamma, beta], axis=0)
    w2row = W2.reshape(1, 128)
    b2m = b2.reshape(1, 1)
    return _mlp2(z, stats, gb, w2row, b2m)


# speedup vs baseline: 7.6154x; 1.0033x over previous
import functools

import jax
import jax.numpy as jnp
from jax import lax
from jax.experimental import pallas as pl
from jax.experimental.pallas import tpu as pltpu
from jax.experimental.pallas import tpu_sc as plsc

B = 16384
F = 26
DENSE = 13
V = 100000
H = 32
EMB_W = F * H
LOOKUPS = B * F

NC = 2
NS = 16
NW = NC * NS
CHUNK = 128
CPW = B // CHUNK // NW

BM = 1024
NB = B // BM


def _sc_gather(tables, idxT):
    mesh = plsc.VectorSubcoreMesh(core_axis_name="c", subcore_axis_name="s")

    @functools.partial(
        pl.kernel,
        mesh=mesh,
        compiler_params=pltpu.CompilerParams(use_tc_tiling_on_sc=False),
        out_type=jax.ShapeDtypeStruct((B, EMB_W), jnp.float32),
        scratch_types=[
            pltpu.VMEM((F, CPW, CHUNK), jnp.int32),
            pltpu.VMEM((CHUNK, H), jnp.float32),
            pltpu.SemaphoreType.DMA,
        ],
    )
    def k(tab_hbm, idx_hbm, out_hbm, idx_v, rows, sem):
        wid = lax.axis_index("s") * NC + lax.axis_index("c")
        pltpu.sync_copy(idx_hbm.at[:, pl.ds(wid * CPW, CPW)], idx_v)

        def task(t, carry):
            f = t // CPW
            j = t % CPW
            c = wid * CPW + j
            pltpu.async_copy(tab_hbm.at[f].at[idx_v.at[f, j]], rows, sem).wait()
            pltpu.sync_copy(
                rows, out_hbm.at[pl.ds(c * CHUNK, CHUNK), pl.ds(f * H, H)])
            return carry

        lax.fori_loop(0, F * CPW, task, 0)

    return k(tables, idxT)


def _mlp1_body(emb_ref, xd_ref, w1e_ref, w1d_ref, z_ref, stats_ref):
    j = pl.program_id(0)
    z = jnp.dot(emb_ref[...], w1e_ref[...], preferred_element_type=jnp.float32)
    z = z + jnp.dot(xd_ref[...], w1d_ref[...], preferred_element_type=jnp.float32)
    z_ref[...] = z
    s1 = jnp.sum(z, axis=0, keepdims=True)
    s2 = jnp.sum(z * z, axis=0, keepdims=True)

    @pl.when(j == 0)
    def _():
        stats_ref[...] = jnp.zeros_like(stats_ref)

    stats_ref[...] += jnp.concatenate(
        [s1, s2, jnp.zeros((6, 128), jnp.float32)], axis=0)


def _mlp1(emb, xdp, w1e, w1dp):
    return pl.pallas_call(
        _mlp1_body,
        grid=(NB,),
        in_specs=[
            pl.BlockSpec((BM, EMB_W), lambda j: (j, 0)),
            pl.BlockSpec((BM, 16), lambda j: (j, 0)),
            pl.BlockSpec((EMB_W, 128), lambda j: (0, 0)),
            pl.BlockSpec((16, 128), lambda j: (0, 0)),
        ],
        out_specs=[
            pl.BlockSpec((BM, 128), lambda j: (j, 0)),
            pl.BlockSpec((8, 128), lambda j: (0, 0)),
        ],
        out_shape=[
            jax.ShapeDtypeStruct((B, 128), jnp.float32),
            jax.ShapeDtypeStruct((8, 128), jnp.float32),
        ],
    )(emb, xdp, w1e, w1dp)


def _mlp2_body(z_ref, stats_ref, gb_ref, w2_ref, b2_ref, out_ref):
    stats = stats_ref[...]
    mean = stats[0:1] / B
    var = stats[1:2] / B - mean * mean
    scale = gb_ref[0:1] * lax.rsqrt(var + 1e-5)
    shift = gb_ref[1:2] - mean * scale
    a = jnp.maximum(z_ref[...] * scale + shift, 0.0)
    o = jnp.sum(a * w2_ref[...], axis=1, keepdims=True) + b2_ref[...]
    out_ref[...] = jax.nn.sigmoid(o)


def _mlp2(z, stats, gb, w2row, b2):
    return pl.pallas_call(
        _mlp2_body,
        grid=(NB,),
        in_specs=[
            pl.BlockSpec((BM, 128), lambda j: (j, 0)),
            pl.BlockSpec((8, 128), lambda j: (0, 0)),
            pl.BlockSpec((2, 128), lambda j: (0, 0)),
            pl.BlockSpec((1, 128), lambda j: (0, 0)),
            pl.BlockSpec((1, 1), lambda j: (0, 0)),
        ],
        out_specs=pl.BlockSpec((BM, 1), lambda j: (j, 0)),
        out_shape=jax.ShapeDtypeStruct((B, 1), jnp.float32),
    )(z, stats, gb, w2row, b2)


def kernel(x, tables, W1, gamma, beta, W2, b2):
    idxT = x[:, :F].astype(jnp.int32).T.reshape(F, NW * CPW, CHUNK)

    emb = _sc_gather(tables, idxT)

    xdp = jnp.pad(x[:, F:], ((0, 0), (0, 16 - DENSE)))
    w1e = W1[:EMB_W]
    w1dp = jnp.pad(W1[EMB_W:], ((0, 16 - DENSE), (0, 0)))
    z, stats = _mlp1(emb, xdp, w1e, w1dp)

    gb = jnp.stack([gamma, beta], axis=0)
    w2row = W2.reshape(1, 128)
    b2m = b2.reshape(1, 1)
    return _mlp2(z, stats, gb, w2row, b2m)
